# baseline (speedup 1.0000x reference)
import functools

import jax
import jax.numpy as jnp
from jax import lax
from jax.experimental import pallas as pl
from jax.experimental.pallas import tpu as pltpu
from jax.experimental.pallas import tpu_sc as plsc

N_NODES = 10000
N_EDGES = 320000
D = 128
K_HOPS = 3

NC = 2
NS = 16
NW = NC * NS
C = 80
Q = 3
L = 2
EPW = N_EDGES // NW
CH = EPW // C
BLK = 5
NB = CH // BLK
RPS = 624
ZR = 8
TAIL = N_NODES - NS * RPS

assert N_EDGES == NW * CH * C and CH == NB * BLK and BLK >= L and Q > L
assert RPS % ZR == 0 and TAIL == 16 and N_NODES % 16 == 0

_mesh = plsc.VectorSubcoreMesh(
    core_axis_name="c", subcore_axis_name="s", num_cores=NC, num_subcores=NS
)


@functools.partial(
    pl.kernel,
    out_type=jax.ShapeDtypeStruct((NW, 1, N_NODES), jnp.float32),
    mesh=_mesh,
    scratch_types=[
        pltpu.VMEM((CH, C), jnp.int32),
        pltpu.VMEM((N_NODES,), jnp.float32),
    ],
    compiler_params=pltpu.CompilerParams(needs_layout_passes=False),
)
def _deg_kernel(dst_hbm, out_hbm, dst_v, cnt_v):
    c = lax.axis_index("c")
    s = lax.axis_index("s")
    wid = c * NS + s
    zero16 = jnp.zeros((16,), jnp.float32)

    def zbody(i, carry):
        cnt_v[pl.ds(i * 16, 16)] = zero16
        return carry

    lax.fori_loop(0, N_NODES // 16, zbody, 0)

    pltpu.sync_copy(dst_hbm.at[wid], dst_v)
    ones16 = jnp.ones((16,), jnp.float32)

    def ebody(i, carry):
        for j in range(C // 16):
            idx = dst_v[i, pl.ds(j * 16, 16)]
            plsc.addupdate_scatter(cnt_v, [idx], ones16)
        return carry

    lax.fori_loop(0, CH, ebody, 0)
    pltpu.sync_copy(cnt_v, out_hbm.at[wid, 0])


@functools.partial(
    pl.kernel,
    out_type=jax.ShapeDtypeStruct((NC, N_NODES, D), jnp.float32),
    mesh=_mesh,
    scratch_types=[
        pltpu.VMEM((2, BLK, C), jnp.int32),
        pltpu.VMEM((2, BLK, C), jnp.int32),
        pltpu.VMEM((Q, C, D), jnp.float32),
        pltpu.VMEM((ZR, D), jnp.float32),
        pltpu.VMEM_SHARED((N_NODES, D), jnp.float32),
        pltpu.SemaphoreType.DMA((Q,)),
        pltpu.SemaphoreType.DMA((Q,)),
        pltpu.SemaphoreType.DMA,
        pltpu.SemaphoreType.DMA,
    ],
)
def _hop_kernel(
    g_hbm, src_hbm, dst_hbm, out_hbm,
    src_b, dst_b, rows_v, zbuf_v, acc_sh, sem_g, sem_s, sem_is, sem_id,
):
    c = lax.axis_index("c")
    s = lax.axis_index("s")
    wid = c * NS + s
    zero16 = jnp.zeros((16,), jnp.float32)

    pltpu.sync_copy(src_hbm.at[wid, 0], src_b.at[0])
    pltpu.sync_copy(dst_hbm.at[wid, 0], dst_b.at[0])
    for j in range(L):
        pltpu.async_copy(g_hbm.at[src_b.at[0, j]], rows_v.at[j], sem_g.at[j])

    def zbody(i, carry):
        for j in range(D // 16):
            zbuf_v[i, pl.ds(j * 16, 16)] = zero16
        return carry

    lax.fori_loop(0, ZR, zbody, 0)
    base = pl.multiple_of(s * RPS, 8)
    for r in range(RPS // ZR):
        pltpu.sync_copy(zbuf_v, acc_sh.at[pl.ds(base + r * ZR, ZR), :])

    @pl.when(s == NS - 1)
    def _zero_tail():
        pltpu.sync_copy(
            zbuf_v.at[pl.ds(0, TAIL), :], acc_sh.at[pl.ds(NS * RPS, TAIL), :]
        )

    plsc.subcore_barrier()

    def ebody(i, carry):
        b = i // BLK
        r = i % BLK
        bp = b & 1
        p = lax.rem(i, Q)
        pn = lax.rem(i + L, Q)
        pltpu.make_async_copy(
            g_hbm.at[src_b.at[bp, r]], rows_v.at[p], sem_g.at[p]
        ).wait()

        @pl.when(i >= Q - L)
        def _wait_scatter():
            pltpu.make_async_copy(
                rows_v.at[pn], acc_sh.at[pl.ds(0, C), :], sem_s.at[pn]
            ).wait()

        @pl.when((r == 0) & (b + 1 < NB))
        def _prefetch_src_idx():
            pltpu.async_copy(src_hbm.at[wid, b + 1], src_b.at[1 - bp], sem_is)

        @pl.when((r == Q - L) & (b + 1 < NB))
        def _prefetch_dst_idx():
            pltpu.async_copy(dst_hbm.at[wid, b + 1], dst_b.at[1 - bp], sem_id)

        @pl.when((r == BLK - L) & (b + 1 < NB))
        def _drain_src_idx():
            pltpu.make_async_copy(src_hbm.at[wid, 0], src_b.at[0], sem_is).wait()

        @pl.when((r == 0) & (b > 0))
        def _drain_dst_idx():
            pltpu.make_async_copy(dst_hbm.at[wid, 0], dst_b.at[0], sem_id).wait()

        @pl.when(i + L < CH)
        def _issue_next():
            i2 = i + L
            b2 = i2 // BLK
            pltpu.async_copy(
                g_hbm.at[src_b.at[b2 & 1, lax.rem(i2, BLK)]],
                rows_v.at[pn],
                sem_g.at[pn],
            )

        pltpu.async_copy(
            rows_v.at[p], acc_sh.at[dst_b.at[bp, r]], sem_s.at[p], add=True
        )
        return carry

    lax.fori_loop(0, CH, ebody, 0)
    for j in range(Q - L):
        pj = (CH - 1 - j) % Q
        pltpu.make_async_copy(
            rows_v.at[pj], acc_sh.at[pl.ds(0, C), :], sem_s.at[pj]
        ).wait()
    plsc.subcore_barrier()
    pltpu.sync_copy(
        acc_sh.at[pl.ds(base, RPS), :], out_hbm.at[c, pl.ds(base, RPS), :]
    )

    @pl.when(s == NS - 1)
    def _copy_tail():
        pltpu.sync_copy(
            acc_sh.at[pl.ds(NS * RPS, TAIL), :],
            out_hbm.at[c, pl.ds(NS * RPS, TAIL), :],
        )


def _reduce_deg(deg_part):
    def body(dp_ref, dinv_ref, dinv2_ref):
        deg = jnp.sum(dp_ref[...], axis=0, keepdims=True) + 1.0
        dinv_ref[...] = lax.rsqrt(deg)
        dinv2_ref[...] = 1.0 / deg

    return pl.pallas_call(
        body,
        out_shape=[
            jax.ShapeDtypeStruct((1, N_NODES), jnp.float32),
            jax.ShapeDtypeStruct((1, N_NODES), jnp.float32),
        ],
    )(deg_part)


def _scale_rows(scale_col, *mats):
    BR = 2000

    def body(*refs):
        s_ref, *in_refs, o_ref = refs
        acc = in_refs[0][...]
        for r in in_refs[1:]:
            acc = acc + r[...]
        o_ref[...] = acc * s_ref[...]

    return pl.pallas_call(
        body,
        grid=(N_NODES // BR,),
        in_specs=[pl.BlockSpec((BR, 1), lambda i: (i, 0))]
        + [pl.BlockSpec((BR, D), lambda i: (i, 0)) for _ in mats],
        out_specs=pl.BlockSpec((BR, D), lambda i: (i, 0)),
        out_shape=jax.ShapeDtypeStruct((N_NODES, D), jnp.float32),
    )(scale_col, *mats)


def _final(p0, p1, g, dinv_col, wt, b2):
    BR = 2000

    def body(a_ref, b_ref, c_ref, s_ref, w_ref, bias_ref, o_ref):
        h = (a_ref[...] + b_ref[...] + c_ref[...]) * s_ref[...]
        o_ref[...] = (
            jnp.dot(h, w_ref[...], preferred_element_type=jnp.float32) + bias_ref[...]
        )

    return pl.pallas_call(
        body,
        grid=(N_NODES // BR,),
        in_specs=[
            pl.BlockSpec((BR, D), lambda i: (i, 0)),
            pl.BlockSpec((BR, D), lambda i: (i, 0)),
            pl.BlockSpec((BR, D), lambda i: (i, 0)),
            pl.BlockSpec((BR, 1), lambda i: (i, 0)),
            pl.BlockSpec((D, D), lambda i: (0, 0)),
            pl.BlockSpec((1, D), lambda i: (0, 0)),
        ],
        out_specs=pl.BlockSpec((BR, D), lambda i: (i, 0)),
        out_shape=jax.ShapeDtypeStruct((N_NODES, D), jnp.float32),
    )(p0, p1, g, dinv_col, wt, b2)


def kernel(V, E, X, W, b):
    del V
    n, d = X.shape
    assert (n, d) == (N_NODES, D) and E.shape == (2, N_EDGES)
    dst3 = E[1].reshape(NW, CH, C)
    src4 = E[0].reshape(NW, NB, BLK, C)
    dst4 = E[1].reshape(NW, NB, BLK, C)

    deg_part = _deg_kernel(dst3).reshape(NW, N_NODES)
    dinv_row, dinv2_row = _reduce_deg(deg_part)
    dinv_col = dinv_row.reshape(N_NODES, 1)
    dinv2_col = dinv2_row.reshape(N_NODES, 1)

    g = _scale_rows(dinv_col, X)
    out = None
    for k in range(K_HOPS):
        p = _hop_kernel(g, src4, dst4)
        if k < K_HOPS - 1:
            g = _scale_rows(dinv2_col, p[0], p[1], g)
        else:
            out = _final(p[0], p[1], g, dinv_col, W.T, b.reshape(1, D))
    return out


# speedup vs baseline: 1.0317x; 1.0317x over previous
import functools

import jax
import jax.numpy as jnp
from jax import lax
from jax.experimental import pallas as pl
from jax.experimental.pallas import tpu as pltpu
from jax.experimental.pallas import tpu_sc as plsc

N_NODES = 10000
N_EDGES = 320000
D = 128
K_HOPS = 3

NC = 2
NS = 16
NW = NC * NS
C = 80
Q = 3
L = 2
EPW = N_EDGES // NW
CH = EPW // C
BLK = 5
NB = CH // BLK
RPS = 624
ZR = 16
TAIL = N_NODES - NS * RPS

assert N_EDGES == NW * CH * C and CH == NB * BLK and BLK >= L and Q > L
assert RPS % ZR == 0 and TAIL == 16 and N_NODES % 16 == 0

_mesh = plsc.VectorSubcoreMesh(
    core_axis_name="c", subcore_axis_name="s", num_cores=NC, num_subcores=NS
)


@functools.partial(
    pl.kernel,
    out_type=jax.ShapeDtypeStruct((NW, 1, N_NODES), jnp.float32),
    mesh=_mesh,
    scratch_types=[
        pltpu.VMEM((CH, C), jnp.int32),
        pltpu.VMEM((N_NODES,), jnp.float32),
    ],
    compiler_params=pltpu.CompilerParams(needs_layout_passes=False),
)
def _deg_kernel(dst_hbm, out_hbm, dst_v, cnt_v):
    c = lax.axis_index("c")
    s = lax.axis_index("s")
    wid = c * NS + s
    zero16 = jnp.zeros((16,), jnp.float32)

    def zbody(i, carry):
        cnt_v[pl.ds(i * 16, 16)] = zero16
        return carry

    lax.fori_loop(0, N_NODES // 16, zbody, 0)

    pltpu.sync_copy(dst_hbm.at[wid], dst_v)
    ones16 = jnp.ones((16,), jnp.float32)

    def ebody(i, carry):
        for j in range(C // 16):
            idx = dst_v[i, pl.ds(j * 16, 16)]
            plsc.addupdate_scatter(cnt_v, [idx], ones16)
        return carry

    lax.fori_loop(0, CH, ebody, 0)
    pltpu.sync_copy(cnt_v, out_hbm.at[wid, 0])


@functools.partial(
    pl.kernel,
    out_type=jax.ShapeDtypeStruct((NC, N_NODES, D), jnp.float32),
    mesh=_mesh,
    scratch_types=[
        pltpu.VMEM((2, BLK, C), jnp.int32),
        pltpu.VMEM((2, BLK, C), jnp.int32),
        pltpu.VMEM((Q, C, D), jnp.float32),
        pltpu.VMEM((ZR, D), jnp.float32),
        pltpu.VMEM_SHARED((N_NODES, D), jnp.float32),
        pltpu.SemaphoreType.DMA((Q,)),
        pltpu.SemaphoreType.DMA((Q,)),
        pltpu.SemaphoreType.DMA,
        pltpu.SemaphoreType.DMA,
    ],
)
def _hop_kernel(
    g_hbm, src_hbm, dst_hbm, out_hbm,
    src_b, dst_b, rows_v, zbuf_v, acc_sh, sem_g, sem_s, sem_is, sem_id,
):
    c = lax.axis_index("c")
    s = lax.axis_index("s")
    wid = c * NS + s
    zero16 = jnp.zeros((16,), jnp.float32)

    pltpu.sync_copy(src_hbm.at[wid, 0], src_b.at[0])
    pltpu.sync_copy(dst_hbm.at[wid, 0], dst_b.at[0])
    for j in range(L):
        pltpu.async_copy(g_hbm.at[src_b.at[0, j]], rows_v.at[j], sem_g.at[j])

    def zbody(i, carry):
        for j in range(D // 16):
            zbuf_v[i, pl.ds(j * 16, 16)] = zero16
        return carry

    lax.fori_loop(0, ZR, zbody, 0)
    base = pl.multiple_of(s * RPS, 8)
    for r in range(RPS // ZR):
        pltpu.async_copy(zbuf_v, acc_sh.at[pl.ds(base + r * ZR, ZR), :], sem_is)

    @pl.when(s == NS - 1)
    def _zero_tail():
        pltpu.sync_copy(
            zbuf_v.at[pl.ds(0, TAIL), :], acc_sh.at[pl.ds(NS * RPS, TAIL), :]
        )

    for r in range(RPS // ZR):
        pltpu.make_async_copy(
            zbuf_v, acc_sh.at[pl.ds(base + r * ZR, ZR), :], sem_is
        ).wait()
    plsc.subcore_barrier()

    def ebody(i, carry):
        b = i // BLK
        r = i % BLK
        bp = b & 1
        p = lax.rem(i, Q)
        pn = lax.rem(i + L, Q)
        pltpu.make_async_copy(
            g_hbm.at[src_b.at[bp, r]], rows_v.at[p], sem_g.at[p]
        ).wait()

        @pl.when(i >= Q - L)
        def _wait_scatter():
            pltpu.make_async_copy(
                rows_v.at[pn], acc_sh.at[pl.ds(0, C), :], sem_s.at[pn]
            ).wait()

        @pl.when((r == 0) & (b + 1 < NB))
        def _prefetch_src_idx():
            pltpu.async_copy(src_hbm.at[wid, b + 1], src_b.at[1 - bp], sem_is)

        @pl.when((r == Q - L) & (b + 1 < NB))
        def _prefetch_dst_idx():
            pltpu.async_copy(dst_hbm.at[wid, b + 1], dst_b.at[1 - bp], sem_id)

        @pl.when((r == BLK - L) & (b + 1 < NB))
        def _drain_src_idx():
            pltpu.make_async_copy(src_hbm.at[wid, 0], src_b.at[0], sem_is).wait()

        @pl.when((r == 0) & (b > 0))
        def _drain_dst_idx():
            pltpu.make_async_copy(dst_hbm.at[wid, 0], dst_b.at[0], sem_id).wait()

        @pl.when(i + L < CH)
        def _issue_next():
            i2 = i + L
            b2 = i2 // BLK
            pltpu.async_copy(
                g_hbm.at[src_b.at[b2 & 1, lax.rem(i2, BLK)]],
                rows_v.at[pn],
                sem_g.at[pn],
            )

        pltpu.async_copy(
            rows_v.at[p], acc_sh.at[dst_b.at[bp, r]], sem_s.at[p], add=True
        )
        return carry

    lax.fori_loop(0, CH, ebody, 0)
    for j in range(Q - L):
        pj = (CH - 1 - j) % Q
        pltpu.make_async_copy(
            rows_v.at[pj], acc_sh.at[pl.ds(0, C), :], sem_s.at[pj]
        ).wait()
    plsc.subcore_barrier()
    pltpu.sync_copy(
        acc_sh.at[pl.ds(base, RPS), :], out_hbm.at[c, pl.ds(base, RPS), :]
    )

    @pl.when(s == NS - 1)
    def _copy_tail():
        pltpu.sync_copy(
            acc_sh.at[pl.ds(NS * RPS, TAIL), :],
            out_hbm.at[c, pl.ds(NS * RPS, TAIL), :],
        )


def _reduce_deg(deg_part):
    def body(dp_ref, dinv_ref, dinv2_ref):
        deg = jnp.sum(dp_ref[...], axis=0, keepdims=True) + 1.0
        dinv_ref[...] = lax.rsqrt(deg)
        dinv2_ref[...] = 1.0 / deg

    return pl.pallas_call(
        body,
        out_shape=[
            jax.ShapeDtypeStruct((1, N_NODES), jnp.float32),
            jax.ShapeDtypeStruct((1, N_NODES), jnp.float32),
        ],
    )(deg_part)


def _scale_rows(scale_col, *mats):
    BR = 2000

    def body(*refs):
        s_ref, *in_refs, o_ref = refs
        acc = in_refs[0][...]
        for r in in_refs[1:]:
            acc = acc + r[...]
        o_ref[...] = acc * s_ref[...]

    return pl.pallas_call(
        body,
        grid=(N_NODES // BR,),
        in_specs=[pl.BlockSpec((BR, 1), lambda i: (i, 0))]
        + [pl.BlockSpec((BR, D), lambda i: (i, 0)) for _ in mats],
        out_specs=pl.BlockSpec((BR, D), lambda i: (i, 0)),
        out_shape=jax.ShapeDtypeStruct((N_NODES, D), jnp.float32),
    )(scale_col, *mats)


def _final(p0, p1, g, dinv_col, wt, b2):
    BR = 2000

    def body(a_ref, b_ref, c_ref, s_ref, w_ref, bias_ref, o_ref):
        h = (a_ref[...] + b_ref[...] + c_ref[...]) * s_ref[...]
        o_ref[...] = (
            jnp.dot(h, w_ref[...], preferred_element_type=jnp.float32) + bias_ref[...]
        )

    return pl.pallas_call(
        body,
        grid=(N_NODES // BR,),
        in_specs=[
            pl.BlockSpec((BR, D), lambda i: (i, 0)),
            pl.BlockSpec((BR, D), lambda i: (i, 0)),
            pl.BlockSpec((BR, D), lambda i: (i, 0)),
            pl.BlockSpec((BR, 1), lambda i: (i, 0)),
            pl.BlockSpec((D, D), lambda i: (0, 0)),
            pl.BlockSpec((1, D), lambda i: (0, 0)),
        ],
        out_specs=pl.BlockSpec((BR, D), lambda i: (i, 0)),
        out_shape=jax.ShapeDtypeStruct((N_NODES, D), jnp.float32),
    )(p0, p1, g, dinv_col, wt, b2)


def kernel(V, E, X, W, b):
    del V
    n, d = X.shape
    assert (n, d) == (N_NODES, D) and E.shape == (2, N_EDGES)
    dst3 = E[1].reshape(NW, CH, C)
    src4 = E[0].reshape(NW, NB, BLK, C)
    dst4 = E[1].reshape(NW, NB, BLK, C)

    deg_part = _deg_kernel(dst3).reshape(NW, N_NODES)
    dinv_row, dinv2_row = _reduce_deg(deg_part)
    dinv_col = dinv_row.reshape(N_NODES, 1)
    dinv2_col = dinv2_row.reshape(N_NODES, 1)

    g = _scale_rows(dinv_col, X)
    out = None
    for k in range(K_HOPS):
        p = _hop_kernel(g, src4, dst4)
        if k < K_HOPS - 1:
            g = _scale_rows(dinv2_col, p[0], p[1], g)
        else:
            out = _final(p[0], p[1], g, dinv_col, W.T, b.reshape(1, D))
    return out


# speedup vs baseline: 1.0462x; 1.0141x over previous
import functools

import jax
import jax.numpy as jnp
from jax import lax
from jax.experimental import pallas as pl
from jax.experimental.pallas import tpu as pltpu
from jax.experimental.pallas import tpu_sc as plsc

N_NODES = 10000
N_EDGES = 320000
D = 128
K_HOPS = 3

NC = 2
NS = 16
NW = NC * NS
C = 80
Q = 3
L = 2
EPW = N_EDGES // NW
CH = EPW // C
BLK = 5
NB = CH // BLK
RPS = 624
ZR = 16
TAIL = N_NODES - NS * RPS

assert N_EDGES == NW * CH * C and CH == NB * BLK and BLK >= L and Q > L
assert RPS % ZR == 0 and TAIL == 16 and N_NODES % 16 == 0

_mesh = plsc.VectorSubcoreMesh(
    core_axis_name="c", subcore_axis_name="s", num_cores=NC, num_subcores=NS
)


@functools.partial(
    pl.kernel,
    out_type=jax.ShapeDtypeStruct((NW, 1, N_NODES), jnp.float32),
    mesh=_mesh,
    scratch_types=[
        pltpu.VMEM((CH, C), jnp.int32),
        pltpu.VMEM((N_NODES,), jnp.float32),
    ],
    compiler_params=pltpu.CompilerParams(needs_layout_passes=False),
)
def _deg_kernel(dst_hbm, out_hbm, dst_v, cnt_v):
    c = lax.axis_index("c")
    s = lax.axis_index("s")
    wid = c * NS + s
    zero16 = jnp.zeros((16,), jnp.float32)

    def zbody(i, carry):
        cnt_v[pl.ds(i * 16, 16)] = zero16
        return carry

    lax.fori_loop(0, N_NODES // 16, zbody, 0)

    pltpu.sync_copy(dst_hbm.at[wid], dst_v)
    ones16 = jnp.ones((16,), jnp.float32)

    def ebody(i, carry):
        for j in range(C // 16):
            idx = dst_v[i, pl.ds(j * 16, 16)]
            plsc.addupdate_scatter(cnt_v, [idx], ones16)
        return carry

    lax.fori_loop(0, CH, ebody, 0)
    pltpu.sync_copy(cnt_v, out_hbm.at[wid, 0])


@functools.partial(
    pl.kernel,
    out_type=jax.ShapeDtypeStruct((NC, N_NODES, D), jnp.float32),
    mesh=_mesh,
    scratch_types=[
        pltpu.VMEM((2, BLK, C), jnp.int32),
        pltpu.VMEM((2, BLK, C), jnp.int32),
        pltpu.VMEM((Q, C, D), jnp.float32),
        pltpu.VMEM((ZR, D), jnp.float32),
        pltpu.VMEM_SHARED((N_NODES, D), jnp.float32),
        pltpu.SemaphoreType.DMA((Q,)),
        pltpu.SemaphoreType.DMA((Q,)),
        pltpu.SemaphoreType.DMA,
        pltpu.SemaphoreType.DMA,
    ],
)
def _hop_kernel(
    g_hbm, src_hbm, dst_hbm, out_hbm,
    src_b, dst_b, rows_v, zbuf_v, acc_sh, sem_g, sem_s, sem_is, sem_id,
):
    c = lax.axis_index("c")
    s = lax.axis_index("s")
    wid = c * NS + s
    zero16 = jnp.zeros((16,), jnp.float32)

    pltpu.sync_copy(src_hbm.at[wid, 0], src_b.at[0])
    pltpu.sync_copy(dst_hbm.at[wid, 0], dst_b.at[0])
    for j in range(L):
        pltpu.async_copy(g_hbm.at[src_b.at[0, j]], rows_v.at[j], sem_g.at[j])

    def zbody(i, carry):
        for j in range(D // 16):
            zbuf_v[i, pl.ds(j * 16, 16)] = zero16
        return carry

    lax.fori_loop(0, ZR, zbody, 0)
    base = pl.multiple_of(s * RPS, 8)
    for r in range(RPS // ZR):
        pltpu.async_copy(zbuf_v, acc_sh.at[pl.ds(base + r * ZR, ZR), :], sem_is)

    @pl.when(s == NS - 1)
    def _zero_tail():
        pltpu.sync_copy(
            zbuf_v.at[pl.ds(0, TAIL), :], acc_sh.at[pl.ds(NS * RPS, TAIL), :]
        )

    for r in range(RPS // ZR):
        pltpu.make_async_copy(
            zbuf_v, acc_sh.at[pl.ds(base + r * ZR, ZR), :], sem_is
        ).wait()
    plsc.subcore_barrier()

    def ebody(i, carry):
        b = i // BLK
        r = i % BLK
        bp = b & 1
        p = lax.rem(i, Q)
        pn = lax.rem(i + L, Q)
        pltpu.make_async_copy(
            g_hbm.at[src_b.at[bp, r]], rows_v.at[p], sem_g.at[p]
        ).wait()

        @pl.when(i >= Q - L)
        def _wait_scatter():
            pltpu.make_async_copy(
                rows_v.at[pn], acc_sh.at[pl.ds(0, C), :], sem_s.at[pn]
            ).wait()

        @pl.when((r == 0) & (b + 1 < NB))
        def _prefetch_src_idx():
            pltpu.async_copy(src_hbm.at[wid, b + 1], src_b.at[1 - bp], sem_is)

        @pl.when((r == Q - L) & (b + 1 < NB))
        def _prefetch_dst_idx():
            pltpu.async_copy(dst_hbm.at[wid, b + 1], dst_b.at[1 - bp], sem_id)

        @pl.when((r == BLK - L) & (b + 1 < NB))
        def _drain_src_idx():
            pltpu.make_async_copy(src_hbm.at[wid, 0], src_b.at[0], sem_is).wait()

        @pl.when((r == 0) & (b > 0))
        def _drain_dst_idx():
            pltpu.make_async_copy(dst_hbm.at[wid, 0], dst_b.at[0], sem_id).wait()

        @pl.when(i + L < CH)
        def _issue_next():
            i2 = i + L
            b2 = i2 // BLK
            pltpu.async_copy(
                g_hbm.at[src_b.at[b2 & 1, lax.rem(i2, BLK)]],
                rows_v.at[pn],
                sem_g.at[pn],
            )

        pltpu.async_copy(
            rows_v.at[p], acc_sh.at[dst_b.at[bp, r]], sem_s.at[p], add=True
        )
        return carry

    lax.fori_loop(0, CH, ebody, 0)
    for j in range(Q - L):
        pj = (CH - 1 - j) % Q
        pltpu.make_async_copy(
            rows_v.at[pj], acc_sh.at[pl.ds(0, C), :], sem_s.at[pj]
        ).wait()
    plsc.subcore_barrier()
    pltpu.sync_copy(
        acc_sh.at[pl.ds(base, RPS), :], out_hbm.at[c, pl.ds(base, RPS), :]
    )

    @pl.when(s == NS - 1)
    def _copy_tail():
        pltpu.sync_copy(
            acc_sh.at[pl.ds(NS * RPS, TAIL), :],
            out_hbm.at[c, pl.ds(NS * RPS, TAIL), :],
        )


def _reduce_deg_scale(deg_part, x):
    def body(dp_ref, x_ref, g0_ref, dinv_ref, dinv2_ref):
        deg = jnp.sum(dp_ref[...], axis=0, keepdims=True) + 1.0
        dinv_col = jnp.transpose(lax.rsqrt(deg))
        dinv_ref[...] = dinv_col
        dinv2_ref[...] = jnp.transpose(1.0 / deg)
        g0_ref[...] = x_ref[...] * dinv_col

    return pl.pallas_call(
        body,
        out_shape=[
            jax.ShapeDtypeStruct((N_NODES, D), jnp.float32),
            jax.ShapeDtypeStruct((N_NODES, 1), jnp.float32),
            jax.ShapeDtypeStruct((N_NODES, 1), jnp.float32),
        ],
    )(deg_part, x)


def _scale_rows(scale_col, *mats):
    BR = 2000

    def body(*refs):
        s_ref, *in_refs, o_ref = refs
        acc = in_refs[0][...]
        for r in in_refs[1:]:
            acc = acc + r[...]
        o_ref[...] = acc * s_ref[...]

    return pl.pallas_call(
        body,
        grid=(N_NODES // BR,),
        in_specs=[pl.BlockSpec((BR, 1), lambda i: (i, 0))]
        + [pl.BlockSpec((BR, D), lambda i: (i, 0)) for _ in mats],
        out_specs=pl.BlockSpec((BR, D), lambda i: (i, 0)),
        out_shape=jax.ShapeDtypeStruct((N_NODES, D), jnp.float32),
    )(scale_col, *mats)


def _final(p0, p1, g, dinv_col, wt, b2):
    BR = 2000

    def body(a_ref, b_ref, c_ref, s_ref, w_ref, bias_ref, o_ref):
        h = (a_ref[...] + b_ref[...] + c_ref[...]) * s_ref[...]
        o_ref[...] = (
            jnp.dot(h, w_ref[...], preferred_element_type=jnp.float32) + bias_ref[...]
        )

    return pl.pallas_call(
        body,
        grid=(N_NODES // BR,),
        in_specs=[
            pl.BlockSpec((BR, D), lambda i: (i, 0)),
            pl.BlockSpec((BR, D), lambda i: (i, 0)),
            pl.BlockSpec((BR, D), lambda i: (i, 0)),
            pl.BlockSpec((BR, 1), lambda i: (i, 0)),
            pl.BlockSpec((D, D), lambda i: (0, 0)),
            pl.BlockSpec((1, D), lambda i: (0, 0)),
        ],
        out_specs=pl.BlockSpec((BR, D), lambda i: (i, 0)),
        out_shape=jax.ShapeDtypeStruct((N_NODES, D), jnp.float32),
    )(p0, p1, g, dinv_col, wt, b2)


def kernel(V, E, X, W, b):
    del V
    n, d = X.shape
    assert (n, d) == (N_NODES, D) and E.shape == (2, N_EDGES)
    dst3 = E[1].reshape(NW, CH, C)
    src4 = E[0].reshape(NW, NB, BLK, C)
    dst4 = E[1].reshape(NW, NB, BLK, C)

    deg_part = _deg_kernel(dst3).reshape(NW, N_NODES)
    g, dinv_col, dinv2_col = _reduce_deg_scale(deg_part, X)
    out = None
    for k in range(K_HOPS):
        p = _hop_kernel(g, src4, dst4)
        if k < K_HOPS - 1:
            g = _scale_rows(dinv2_col, p[0], p[1], g)
        else:
            out = _final(p[0], p[1], g, dinv_col, W.T, b.reshape(1, D))
    return out
